# baseline (speedup 1.0000x reference)
import dataclasses
import functools

import jax
import jax.numpy as jnp
from jax import lax
from jax.experimental import pallas as pl
from jax.experimental.pallas import tpu as pltpu
from jax.experimental.pallas import tpu_sc as plsc

E = 160000
D = 256
NC, NS, L = 2, 16, 16
NW = NC * NS
C = 128
NUM_CHUNKS = E // C


def _sc_edge_dot(x, senders, receivers):
    mesh = plsc.VectorSubcoreMesh(core_axis_name="c", subcore_axis_name="s")
    cp = pltpu.CompilerParams()
    if "needs_layout_passes" in pltpu.CompilerParams.__dataclass_fields__:
        cp = dataclasses.replace(cp, needs_layout_passes=False)

    @functools.partial(
        pl.kernel,
        compiler_params=cp,
        out_type=jax.ShapeDtypeStruct((E,), jnp.float32),
        mesh=mesh,
        scratch_types=[
            pltpu.VMEM((C,), jnp.int32),
            pltpu.VMEM((C,), jnp.int32),
            pltpu.VMEM((C, D), jnp.float32),
            pltpu.VMEM((C, D), jnp.float32),
            pltpu.VMEM((C,), jnp.float32),
            pltpu.SemaphoreType.DMA,
            pltpu.SemaphoreType.DMA,
        ],
    )
    def k(x_hbm, s_hbm, r_hbm, o_hbm, s_v, r_v, xs_v, xr_v, o_v, sem1, sem2):
        wid = lax.axis_index("s") * NC + lax.axis_index("c")
        iota16 = lax.iota(jnp.int32, L)

        @pl.loop(wid, NUM_CHUNKS, step=NW)
        def _(c):
            base = c * C
            pltpu.sync_copy(s_hbm.at[pl.ds(base, C)], s_v)
            pltpu.sync_copy(r_hbm.at[pl.ds(base, C)], r_v)
            cp1 = pltpu.async_copy(x_hbm.at[s_v], xs_v, sem1)
            cp2 = pltpu.async_copy(x_hbm.at[r_v], xr_v, sem2)
            cp1.wait()
            cp2.wait()

            @pl.loop(0, C, step=L)
            def _(e0):
                rows = iota16 + e0

                @pl.loop(0, D, init_carry=jnp.zeros((L,), jnp.float32))
                def acc(d, a):
                    cols = jnp.full((L,), d, jnp.int32)
                    xs = plsc.load_gather(xs_v, [rows, cols])
                    xr = plsc.load_gather(xr_v, [rows, cols])
                    return a + xs * xr

                o_v[pl.ds(e0, L)] = acc

            pltpu.sync_copy(o_v, o_hbm.at[pl.ds(base, C)])

    return k(x, senders, receivers)


def kernel(x, edge_index):
    senders = edge_index[0].astype(jnp.int32)
    receivers = edge_index[1].astype(jnp.int32)
    he = _sc_edge_dot(x, senders, receivers)
    return he.reshape(E, 1)


# speedup vs baseline: 1.0432x; 1.0432x over previous
import dataclasses
import functools

import jax
import jax.numpy as jnp
from jax import lax
from jax.experimental import pallas as pl
from jax.experimental.pallas import tpu as pltpu
from jax.experimental.pallas import tpu_sc as plsc

E = 160000
D = 256
NC, NS, L = 2, 16, 16
NW = NC * NS
C = 64
NUM_CHUNKS = E // C
G = -(-NUM_CHUNKS // NW)


def _sc_edge_dot(x, senders, receivers):
    mesh = plsc.VectorSubcoreMesh(core_axis_name="c", subcore_axis_name="s")
    cp = pltpu.CompilerParams()
    if "needs_layout_passes" in pltpu.CompilerParams.__dataclass_fields__:
        cp = dataclasses.replace(cp, needs_layout_passes=False)

    @functools.partial(
        pl.kernel,
        compiler_params=cp,
        out_type=jax.ShapeDtypeStruct((E,), jnp.float32),
        mesh=mesh,
        scratch_types=[
            [pltpu.VMEM((C,), jnp.int32) for _ in range(2)],
            [pltpu.VMEM((C,), jnp.int32) for _ in range(2)],
            [pltpu.VMEM((C, D), jnp.float32) for _ in range(2)],
            [pltpu.VMEM((C, D), jnp.float32) for _ in range(2)],
            pltpu.VMEM((C,), jnp.float32),
            [pltpu.SemaphoreType.DMA for _ in range(2)],
        ],
    )
    def k(x_hbm, s_hbm, r_hbm, o_hbm, s_v, r_v, xs_v, xr_v, o_v, sem):
        wid = lax.axis_index("s") * NC + lax.axis_index("c")
        iota16 = lax.iota(jnp.int32, L)

        def issue(g, b):
            c = wid + g * NW
            base = c * C
            pltpu.sync_copy(s_hbm.at[pl.ds(base, C)], s_v[b])
            pltpu.sync_copy(r_hbm.at[pl.ds(base, C)], r_v[b])
            pltpu.async_copy(x_hbm.at[s_v[b]], xs_v[b], sem[b])
            pltpu.async_copy(x_hbm.at[r_v[b]], xr_v[b], sem[b])

        def drain(b):
            pltpu.make_async_copy(x_hbm.at[s_v[b]], xs_v[b], sem[b]).wait()
            pltpu.make_async_copy(x_hbm.at[r_v[b]], xr_v[b], sem[b]).wait()

        def compute_store(g, b):
            c = wid + g * NW
            base = c * C
            drain(b)

            @pl.loop(0, C, step=L)
            def _(e0):
                rows = iota16 + e0

                @pl.loop(0, D, init_carry=jnp.zeros((L,), jnp.float32),
                         unroll=8)
                def acc(d, a):
                    cols = jnp.full((L,), d, jnp.int32)
                    xs = plsc.load_gather(xs_v[b], [rows, cols])
                    xr = plsc.load_gather(xr_v[b], [rows, cols])
                    return a + xs * xr

                o_v[pl.ds(e0, L)] = acc

            pltpu.sync_copy(o_v, o_hbm.at[pl.ds(base, C)])

        def has_chunk(g):
            return wid + g * NW < NUM_CHUNKS

        pl.when(has_chunk(0))(lambda: issue(0, 0))

        @pl.loop(0, G, step=2)
        def _(g):
            pl.when(has_chunk(g + 1))(lambda: issue(g + 1, 1))
            pl.when(has_chunk(g))(lambda: compute_store(g, 0))
            pl.when(has_chunk(g + 2))(lambda: issue(g + 2, 0))
            pl.when(has_chunk(g + 1))(lambda: compute_store(g + 1, 1))

    return k(x, senders, receivers)


def kernel(x, edge_index):
    senders = edge_index[0].astype(jnp.int32)
    receivers = edge_index[1].astype(jnp.int32)
    he = _sc_edge_dot(x, senders, receivers)
    return he.reshape(E, 1)


# speedup vs baseline: 1.0809x; 1.0361x over previous
import dataclasses
import functools

import jax
import jax.numpy as jnp
from jax import lax
from jax.experimental import pallas as pl
from jax.experimental.pallas import tpu as pltpu
from jax.experimental.pallas import tpu_sc as plsc

E = 160000
D = 256
NC, NS, L = 2, 16, 16
NW = NC * NS
C = 64
NUM_CHUNKS = E // C
G = -(-NUM_CHUNKS // NW)


def _sc_edge_dot(x, senders, receivers):
    mesh = plsc.VectorSubcoreMesh(core_axis_name="c", subcore_axis_name="s")
    cp = pltpu.CompilerParams()
    if "needs_layout_passes" in pltpu.CompilerParams.__dataclass_fields__:
        cp = dataclasses.replace(cp, needs_layout_passes=False)
    if "use_tc_tiling_on_sc" in pltpu.CompilerParams.__dataclass_fields__:
        cp = dataclasses.replace(cp, use_tc_tiling_on_sc=False)

    @functools.partial(
        pl.kernel,
        compiler_params=cp,
        out_type=jax.ShapeDtypeStruct((E,), jnp.float32),
        mesh=mesh,
        scratch_types=[
            [pltpu.VMEM((C,), jnp.int32) for _ in range(2)],
            [pltpu.VMEM((C,), jnp.int32) for _ in range(2)],
            [pltpu.VMEM((C, D), jnp.float32) for _ in range(2)],
            [pltpu.VMEM((C, D), jnp.float32) for _ in range(2)],
            pltpu.VMEM((C,), jnp.float32),
            [pltpu.SemaphoreType.DMA for _ in range(2)],
        ],
    )
    def k(x_hbm, s_hbm, r_hbm, o_hbm, s_v, r_v, xs_v, xr_v, o_v, sem):
        wid = lax.axis_index("s") * NC + lax.axis_index("c")
        iota16 = lax.iota(jnp.int32, L)

        def issue(g, b):
            c = wid + g * NW
            base = c * C
            pltpu.sync_copy(s_hbm.at[pl.ds(base, C)], s_v[b])
            pltpu.sync_copy(r_hbm.at[pl.ds(base, C)], r_v[b])
            pltpu.async_copy(x_hbm.at[s_v[b]], xs_v[b], sem[b])
            pltpu.async_copy(x_hbm.at[r_v[b]], xr_v[b], sem[b])

        def drain(b):
            pltpu.make_async_copy(x_hbm.at[s_v[b]], xs_v[b], sem[b]).wait()
            pltpu.make_async_copy(x_hbm.at[r_v[b]], xr_v[b], sem[b]).wait()

        def compute_store(g, b):
            c = wid + g * NW
            base = c * C
            drain(b)

            @pl.loop(0, C, step=L)
            def _(e0):
                rows = iota16 + e0
                acc = jnp.zeros((L,), jnp.float32)
                for d in range(D):
                    cols = jnp.full((L,), d, jnp.int32)
                    xs = plsc.load_gather(xs_v[b], [rows, cols])
                    xr = plsc.load_gather(xr_v[b], [rows, cols])
                    acc = acc + xs * xr
                o_v[pl.ds(e0, L)] = acc

            pltpu.sync_copy(o_v, o_hbm.at[pl.ds(base, C)])

        def has_chunk(g):
            return wid + g * NW < NUM_CHUNKS

        pl.when(has_chunk(0))(lambda: issue(0, 0))

        @pl.loop(0, G, step=2)
        def _(g):
            pl.when(has_chunk(g + 1))(lambda: issue(g + 1, 1))
            pl.when(has_chunk(g))(lambda: compute_store(g, 0))
            pl.when(has_chunk(g + 2))(lambda: issue(g + 2, 0))
            pl.when(has_chunk(g + 1))(lambda: compute_store(g + 1, 1))

    return k(x, senders, receivers)


def kernel(x, edge_index):
    senders = edge_index[0].astype(jnp.int32)
    receivers = edge_index[1].astype(jnp.int32)
    he = _sc_edge_dot(x, senders, receivers)
    return he.reshape(E, 1)


# speedup vs baseline: 6.3161x; 5.8434x over previous
import dataclasses
import functools

import jax
import jax.numpy as jnp
from jax import lax
from jax.experimental import pallas as pl
from jax.experimental.pallas import tpu as pltpu
from jax.experimental.pallas import tpu_sc as plsc

E = 160000
D = 256
NC, NS, L = 2, 16, 16
NW = NC * NS
C = 64
NUM_CHUNKS = E // C
G = -(-NUM_CHUNKS // NW)


def _sc_edge_dot(x, senders, receivers):
    mesh = plsc.VectorSubcoreMesh(core_axis_name="c", subcore_axis_name="s")
    cp = pltpu.CompilerParams()
    if "needs_layout_passes" in pltpu.CompilerParams.__dataclass_fields__:
        cp = dataclasses.replace(cp, needs_layout_passes=False)
    if "use_tc_tiling_on_sc" in pltpu.CompilerParams.__dataclass_fields__:
        cp = dataclasses.replace(cp, use_tc_tiling_on_sc=False)

    @functools.partial(
        pl.kernel,
        compiler_params=cp,
        out_type=jax.ShapeDtypeStruct((E,), jnp.float32),
        mesh=mesh,
        scratch_types=[
            [pltpu.VMEM((C,), jnp.int32) for _ in range(2)],
            [pltpu.VMEM((C,), jnp.int32) for _ in range(2)],
            [pltpu.VMEM((C, D), jnp.float32) for _ in range(2)],
            [pltpu.VMEM((C, D), jnp.float32) for _ in range(2)],
            pltpu.VMEM((C,), jnp.float32),
            [pltpu.SemaphoreType.DMA for _ in range(2)],
        ],
    )
    def k(x_hbm, s_hbm, r_hbm, o_hbm, s_v, r_v, xs_v, xr_v, o_v, sem):
        wid = lax.axis_index("s") * NC + lax.axis_index("c")
        iota16 = lax.iota(jnp.int32, L)

        def issue(g, b):
            c = wid + g * NW
            base = c * C
            pltpu.sync_copy(s_hbm.at[pl.ds(base, C)], s_v[b])
            pltpu.sync_copy(r_hbm.at[pl.ds(base, C)], r_v[b])
            pltpu.async_copy(x_hbm.at[s_v[b]], xs_v[b], sem[b])
            pltpu.async_copy(x_hbm.at[r_v[b]], xr_v[b], sem[b])

        def drain(b):
            pltpu.make_async_copy(x_hbm.at[s_v[b]], xs_v[b], sem[b]).wait()
            pltpu.make_async_copy(x_hbm.at[r_v[b]], xr_v[b], sem[b]).wait()

        def compute_store(g, b):
            c = wid + g * NW
            base = c * C
            drain(b)

            @pl.loop(0, C, step=L)
            def _(e0):
                rows = iota16 + e0
                acc = jnp.zeros((L,), jnp.float32)
                for d in range(D):
                    cols = iota16 + d
                    if d > D - L:
                        cols = jnp.where(cols >= D, cols - D, cols)
                    xs = plsc.load_gather(xs_v[b], [rows, cols])
                    xr = plsc.load_gather(xr_v[b], [rows, cols])
                    acc = acc + xs * xr
                o_v[pl.ds(e0, L)] = acc

            pltpu.sync_copy(o_v, o_hbm.at[pl.ds(base, C)])

        def has_chunk(g):
            return wid + g * NW < NUM_CHUNKS

        pl.when(has_chunk(0))(lambda: issue(0, 0))

        @pl.loop(0, G, step=2)
        def _(g):
            pl.when(has_chunk(g + 1))(lambda: issue(g + 1, 1))
            pl.when(has_chunk(g))(lambda: compute_store(g, 0))
            pl.when(has_chunk(g + 2))(lambda: issue(g + 2, 0))
            pl.when(has_chunk(g + 1))(lambda: compute_store(g + 1, 1))

    return k(x, senders, receivers)


def kernel(x, edge_index):
    senders = edge_index[0].astype(jnp.int32)
    receivers = edge_index[1].astype(jnp.int32)
    he = _sc_edge_dot(x, senders, receivers)
    return he.reshape(E, 1)
